# Initial kernel scaffold; baseline (speedup 1.0000x reference)
import jax
import jax.numpy as jnp
from jax.experimental import pallas as pl


def kernel(x, edge_index, W, b):
    raise NotImplementedError("write your pallas kernel here")


# speedup vs baseline: 4.6238x; 4.6238x over previous
import functools

import jax
import jax.numpy as jnp
from jax import lax
from jax.experimental import pallas as pl
from jax.experimental.pallas import tpu as pltpu
from jax.experimental.pallas import tpu_sc as plsc

C = 128
N = 10000
K = 32
OUT = 128
GROUPS = 4

NC = 2
NS = 16
NW = NC * NS
CPT = C // NW
NB = 400
NBLK = N // NB
GPB = NB // 16

@functools.cache
def _build_sc_aggregate():
    mesh = plsc.VectorSubcoreMesh(
        core_axis_name="c", subcore_axis_name="s",
        num_cores=NC, num_subcores=NS)
    return pl.kernel(
        _sc_aggregate_body,
        out_type=jax.ShapeDtypeStruct((C, N), jnp.float32),
        mesh=mesh,
        compiler_params=pltpu.CompilerParams(
            use_tc_tiling_on_sc=False, needs_layout_passes=False),
        scratch_types=[
            pltpu.VMEM((CPT, N), jnp.float32),
            pltpu.VMEM((K, NB), jnp.int32),
            pltpu.VMEM((K, NB), jnp.int32),
            pltpu.VMEM((CPT, NB), jnp.float32),
        ],
    )


def _sc_aggregate_body(x_hbm, e0_hbm, e1_hbm, m_hbm, x_v, e0_v, e1_v, m_v):
    wid = lax.axis_index("s") * NC + lax.axis_index("c")
    c0 = wid * CPT
    pltpu.sync_copy(x_hbm.at[pl.ds(c0, CPT), :], x_v)

    csplat = [jnp.full((16,), c, jnp.int32) for c in range(CPT)]

    def blk_body(blk, _):
        nb0 = blk * NB
        pltpu.sync_copy(e0_hbm.at[:, pl.ds(nb0, NB)], e0_v)
        pltpu.sync_copy(e1_hbm.at[:, pl.ds(nb0, NB)], e1_v)

        def g_body(g, _):
            base = g * 16
            vj0 = e0_v[0, pl.ds(base, 16)]
            vi0 = e1_v[0, pl.ds(base, 16)]
            accs = []
            for c in range(CPT):
                xj = plsc.load_gather(x_v, [csplat[c], vj0])
                xi = plsc.load_gather(x_v, [csplat[c], vi0])
                accs.append(xj - xi)
            for k in range(1, K):
                vj = e0_v[k, pl.ds(base, 16)]
                vi = e1_v[k, pl.ds(base, 16)]
                for c in range(CPT):
                    xj = plsc.load_gather(x_v, [csplat[c], vj])
                    xi = plsc.load_gather(x_v, [csplat[c], vi])
                    accs[c] = jnp.maximum(accs[c], xj - xi)
            for c in range(CPT):
                m_v[c, pl.ds(base, 16)] = accs[c]
            return 0

        lax.fori_loop(0, GPB, g_body, 0)
        pltpu.sync_copy(m_v, m_hbm.at[pl.ds(c0, CPT), pl.ds(nb0, NB)])
        return 0

    lax.fori_loop(0, NBLK, blk_body, 0)


def _conv_body(x_ref, m_ref, wx_ref, wm_ref, b_ref, o_ref):
    acc = jnp.dot(wx_ref[...], x_ref[...], preferred_element_type=jnp.float32)
    acc = acc + jnp.dot(wm_ref[...], m_ref[...], preferred_element_type=jnp.float32)
    o_ref[...] = jnp.maximum(acc + b_ref[...], 0.0)


def _grouped_weights(W):
    o = jnp.arange(OUT)
    g = o // (OUT // GROUPS)
    c = jnp.arange(C)
    in_group = (c[None, :] // (C // GROUPS)) == g[:, None]
    a = jnp.clip(c[None, :] - (C // GROUPS) * g[:, None], 0, C // GROUPS - 1)
    Wx = jnp.where(in_group, jnp.take_along_axis(W[:, 0::2], a, axis=1), 0.0)
    Wm = jnp.where(in_group, jnp.take_along_axis(W[:, 1::2], a, axis=1), 0.0)
    return Wx, Wm


def kernel(x, edge_index, W, b):
    xT = x.reshape(C, N)
    e0 = jnp.transpose(edge_index[0].reshape(N, K))
    e1 = jnp.transpose(edge_index[1].reshape(N, K))

    m = _build_sc_aggregate()(xT, e0, e1)

    Wx, Wm = _grouped_weights(W)
    out = pl.pallas_call(
        _conv_body,
        out_shape=jax.ShapeDtypeStruct((OUT, N), jnp.float32),
    )(xT, m, Wx, Wm, b.reshape(OUT, 1))
    return out.reshape(1, OUT, N, 1)


# speedup vs baseline: 4.6822x; 1.0126x over previous
import functools

import jax
import jax.numpy as jnp
from jax import lax
from jax.experimental import pallas as pl
from jax.experimental.pallas import tpu as pltpu
from jax.experimental.pallas import tpu_sc as plsc

C = 128
N = 10000
K = 32
OUT = 128
GROUPS = 4

NC = 2
NS = 16
NW = NC * NS
CPT = C // NW
NB = 400
NBLK = N // NB
GPB = NB // 16


@functools.cache
def _build_sc_aggregate():
    mesh = plsc.VectorSubcoreMesh(
        core_axis_name="c", subcore_axis_name="s",
        num_cores=NC, num_subcores=NS)
    return pl.kernel(
        _sc_aggregate_body,
        out_type=jax.ShapeDtypeStruct((C, N), jnp.float32),
        mesh=mesh,
        compiler_params=pltpu.CompilerParams(
            use_tc_tiling_on_sc=False, needs_layout_passes=False),
        scratch_types=[
            pltpu.VMEM((CPT, N), jnp.float32),
            pltpu.VMEM((CPT, N), jnp.float32),
            pltpu.VMEM((K, NB), jnp.int32),
            pltpu.VMEM((K, NB), jnp.int32),
            pltpu.SemaphoreType.DMA,
            pltpu.SemaphoreType.DMA,
        ],
    )


def _sc_aggregate_body(x_hbm, ep_hbm, m_hbm, x_v, m_v, e_v0, e_v1, sem0, sem1):
    wid = lax.axis_index("s") * NC + lax.axis_index("c")
    c0 = wid * CPT
    pltpu.async_copy(ep_hbm.at[0], e_v0, sem0)
    pltpu.sync_copy(x_hbm.at[pl.ds(c0, CPT), :], x_v)

    csplat = [jnp.full((16,), c, jnp.int32) for c in range(CPT)]

    def compute_block(blk, e_v):
        def g_body(g, _):
            base = blk * NB + g * 16
            lbase = g * 16
            ep = e_v[0, pl.ds(lbase, 16)]
            vj0, vi0 = plsc.unpack(
                plsc.bitcast(ep, jnp.int16), format=plsc.PackFormat.INTERLEAVED)
            accs = []
            for c in range(CPT):
                xj = plsc.load_gather(x_v, [csplat[c], vj0])
                xi = plsc.load_gather(x_v, [csplat[c], vi0])
                accs.append(xj - xi)
            for k in range(1, K):
                ep = e_v[k, pl.ds(lbase, 16)]
                vj, vi = plsc.unpack(
                    plsc.bitcast(ep, jnp.int16),
                    format=plsc.PackFormat.INTERLEAVED)
                for c in range(CPT):
                    xj = plsc.load_gather(x_v, [csplat[c], vj])
                    xi = plsc.load_gather(x_v, [csplat[c], vi])
                    accs[c] = jnp.maximum(accs[c], xj - xi)
            for c in range(CPT):
                m_v[c, pl.ds(base, 16)] = accs[c]
            return 0

        lax.fori_loop(0, GPB, g_body, 0)

    def blk_body(blk, _):
        nxt = blk + 1

        @pl.when(blk % 2 == 0)
        def _even():
            @pl.when(nxt < NBLK)
            def _():
                pltpu.async_copy(ep_hbm.at[nxt], e_v1, sem1)
            pltpu.make_async_copy(ep_hbm.at[blk], e_v0, sem0).wait()
            compute_block(blk, e_v0)

        @pl.when(blk % 2 == 1)
        def _odd():
            @pl.when(nxt < NBLK)
            def _():
                pltpu.async_copy(ep_hbm.at[nxt], e_v0, sem0)
            pltpu.make_async_copy(ep_hbm.at[blk], e_v1, sem1).wait()
            compute_block(blk, e_v1)

        return 0

    lax.fori_loop(0, NBLK, blk_body, 0)
    pltpu.sync_copy(m_v, m_hbm.at[pl.ds(c0, CPT), :])


def _conv_body(x_ref, m_ref, wx_ref, wm_ref, b_ref, o_ref):
    acc = jnp.dot(wx_ref[...], x_ref[...], preferred_element_type=jnp.float32)
    acc = acc + jnp.dot(wm_ref[...], m_ref[...], preferred_element_type=jnp.float32)
    o_ref[...] = jnp.maximum(acc + b_ref[...], 0.0)


def _grouped_weights(W):
    o = jnp.arange(OUT)
    g = o // (OUT // GROUPS)
    c = jnp.arange(C)
    in_group = (c[None, :] // (C // GROUPS)) == g[:, None]
    a = jnp.clip(c[None, :] - (C // GROUPS) * g[:, None], 0, C // GROUPS - 1)
    Wx = jnp.where(in_group, jnp.take_along_axis(W[:, 0::2], a, axis=1), 0.0)
    Wm = jnp.where(in_group, jnp.take_along_axis(W[:, 1::2], a, axis=1), 0.0)
    return Wx, Wm


def kernel(x, edge_index, W, b):
    xT = x.reshape(C, N)
    ep = edge_index[0, 0] | (edge_index[1, 0] << 16)
    ep = jnp.transpose(ep.reshape(NBLK, NB, K), (0, 2, 1))

    m = _build_sc_aggregate()(xT, ep)

    Wx, Wm = _grouped_weights(W)
    out = pl.pallas_call(
        _conv_body,
        out_shape=jax.ShapeDtypeStruct((OUT, N), jnp.float32),
    )(xT, m, Wx, Wm, b.reshape(OUT, 1))
    return out.reshape(1, OUT, N, 1)
